# baseline (speedup 1.0000x reference)
import functools

import jax
import jax.numpy as jnp
import numpy as np
from jax.experimental import pallas as pl
from jax.experimental.pallas import tpu as pltpu

N_RES = 512
N_ATOM = 8192
HIDDEN = 128
NUM_RBF = 16
CUTOFF = 12.0
ATOM_CUTOFF = 5.0
EMAX_RES = 8192
EMAX_ATOM = 262144

EB = 256
WOUT = EB + 8
PCOLS = HIDDEN + 4


def _edge_block_kernel(rb8_ref, nvb_ref, gs_ref, gd_ref, w_ref, out_ref, *,
                       cutoff):
    b = pl.program_id(0)

    @pl.when(b == 0)
    def _init():
        out_ref[...] = jnp.zeros_like(out_ref)

    @pl.when(b < nvb_ref[0])
    def _compute():
        gs = gs_ref[...]
        gd = gd_ref[...]
        a = gs[:, :HIDDEN]
        xs = gs[:, HIDDEN:HIDDEN + 3]
        vm = gs[:, HIDDEN + 3:HIDDEN + 4]
        lid = gs[:, HIDDEN + 4:HIDDEN + 5]
        bv = gd[:, :HIDDEN]
        xd = gd[:, HIDDEN:HIDDEN + 3]

        diff = xd - xs
        d = jnp.sqrt(jnp.sum(diff * diff, axis=1, keepdims=True) + 1e-12)
        centers = jax.lax.broadcasted_iota(
            jnp.int32, (1, NUM_RBF), 1).astype(jnp.float32) \
            * np.float32(cutoff / (NUM_RBF - 1))
        width = cutoff / NUM_RBF
        rbf = jnp.exp(-((d - centers) ** 2) / (2.0 * width * width))

        pre = a + bv + jnp.dot(rbf, w_ref[0:NUM_RBF, :],
                               preferred_element_type=jnp.float32)
        m1 = jax.nn.silu(pre)
        m = jax.nn.silu(
            jnp.dot(m1, w_ref[16:144, :], preferred_element_type=jnp.float32)
            + w_ref[272:273, :]) * vm
        s = jax.nn.silu(
            jnp.dot(m, w_ref[144:272, :], preferred_element_type=jnp.float32)
            + w_ref[273:274, :])
        c = jnp.sum(s * w_ref[274:275, :], axis=1, keepdims=True) \
            + w_ref[275:276, 0:1]
        trans = diff * (c * vm)

        p = jnp.concatenate([m, trans, vm], axis=1)
        sel = (lid == jax.lax.broadcasted_iota(jnp.int32, (EB, WOUT), 1)
               .astype(jnp.float32)).astype(jnp.float32)
        l = jax.lax.dot_general(sel, p, (((0,), (0,)), ((), ())),
                                preferred_element_type=jnp.float32)
        rb = pl.multiple_of(rb8_ref[b], 8)
        out_ref[pl.ds(rb, WOUT), :] += l


def _run_edge_kernel(gs, gd, wpack, rb8, nvb, nseg_pad, cutoff):
    nb = gs.shape[0] // EB
    grid_spec = pltpu.PrefetchScalarGridSpec(
        num_scalar_prefetch=2,
        grid=(nb,),
        in_specs=[
            pl.BlockSpec((EB, gs.shape[1]), lambda b, rb8, nvb: (b, 0)),
            pl.BlockSpec((EB, gd.shape[1]), lambda b, rb8, nvb: (b, 0)),
            pl.BlockSpec(wpack.shape, lambda b, rb8, nvb: (0, 0)),
        ],
        out_specs=pl.BlockSpec((nseg_pad, PCOLS), lambda b, rb8, nvb: (0, 0)),
    )
    return pl.pallas_call(
        functools.partial(_edge_block_kernel, cutoff=cutoff),
        grid_spec=grid_spec,
        out_shape=jax.ShapeDtypeStruct((nseg_pad, PCOLS), jnp.float32),
    )(rb8, nvb, gs, gd, wpack)


def _pack_weights(p):
    cb2 = jnp.broadcast_to(p['cb2'].reshape(1, 1), (1, HIDDEN))
    return jnp.concatenate([
        p['eW1'][2 * HIDDEN:],
        p['eW2'],
        p['cW1'],
        p['eb2'].reshape(1, HIDDEN),
        p['cb1'].reshape(1, HIDDEN),
        p['cW2'].reshape(1, HIDDEN),
        cb2,
        jnp.zeros((4, HIDDEN), jnp.float32),
    ], axis=0)


def _egnn_layer_pallas(h, x, src, dst, valid, p, cutoff, n):
    epad = src.shape[0]
    nb = epad // EB
    nseg_pad = n + WOUT + 8

    a = h @ p['eW1'][:HIDDEN]
    bn = h @ p['eW1'][HIDDEN:2 * HIDDEN] + p['eb1']
    vm = valid.astype(jnp.float32)

    dst32 = dst.astype(jnp.int32)
    is_start = jnp.concatenate([
        jnp.ones((1,), jnp.int32),
        (dst32[1:] != dst32[:-1]).astype(jnp.int32)])
    gseg = jnp.cumsum(is_start) - 1
    row_base = gseg.reshape(nb, EB)[:, 0]
    rb8 = ((row_base // 8) * 8).astype(jnp.int32)
    lid = (gseg.reshape(nb, EB) - rb8[:, None]).reshape(epad)
    seg_rows = jnp.zeros((n,), jnp.int32).at[gseg].set(dst32, mode='drop')
    nvalid = jnp.sum(vm).astype(jnp.int32)
    nvb = ((nvalid + EB - 1) // EB).reshape(1)

    gs = jnp.concatenate([
        a[src], x[src], vm[:, None], lid.astype(jnp.float32)[:, None]], axis=1)
    gd = jnp.concatenate([bn[dst], x[dst]], axis=1)

    out_seg = _run_edge_kernel(gs, gd, _pack_weights(p), rb8, nvb, nseg_pad,
                               cutoff)
    agg = jnp.zeros((n, PCOLS), jnp.float32).at[seg_rows].add(out_seg[:n])

    agg_m = agg[:, :HIDDEN]
    agg_t = agg[:, HIDDEN:HIDDEN + 3]
    cnt = agg[:, HIDDEN + 3:HIDDEN + 4]
    x_new = x + agg_t / (cnt + 1.0)
    h_new = h + jax.nn.silu(
        jnp.concatenate([h, agg_m], axis=1) @ p['nW1'] + p['nb1']
    ) @ p['nW2'] + p['nb2']
    return h_new, x_new


_KMAX = 128


def _atom_edges_sorted(x, cutoff, emax):
    n = x.shape[0]
    sq = jnp.sum(x * x, axis=1)
    d2 = sq[:, None] + sq[None, :] - 2.0 * (x @ x.T)
    mask = (d2 < cutoff * cutoff) & (~jnp.eye(n, dtype=bool))
    cnt = jnp.sum(mask, axis=1).astype(jnp.int32)
    offs = jnp.concatenate(
        [jnp.zeros((1,), jnp.int32), jnp.cumsum(cnt)[:-1].astype(jnp.int32)])
    iota = jnp.arange(n, dtype=jnp.int32)
    vals = jnp.where(mask, iota[None, :], -1)
    topv, _ = jax.lax.top_k(vals, _KMAX)
    kk = jnp.arange(_KMAX, dtype=jnp.int32)[None, :]
    validk = kk < cnt[:, None]
    pos = jnp.where(validk, offs[:, None] + kk, emax)
    srcv = jnp.full((emax,), n - 1, jnp.int32).at[pos.reshape(-1)].set(
        jnp.maximum(topv, 0).reshape(-1), mode='drop')
    dstv = jnp.full((emax,), n - 1, jnp.int32).at[pos.reshape(-1)].set(
        jnp.broadcast_to(iota[:, None], (n, _KMAX)).reshape(-1), mode='drop')
    valid = jnp.arange(emax) < jnp.sum(cnt)
    return srcv, dstv, valid


def _rbf_ref(d, num_rbf, cutoff):
    centers = jnp.linspace(0.0, cutoff, num_rbf)
    width = cutoff / num_rbf
    return jnp.exp(-((d[:, None] - centers[None, :]) ** 2)
                   / (2.0 * width * width))


def _residue_graph_ref(x, cutoff, max_edges):
    n = x.shape[0]
    sq = jnp.sum(x * x, axis=1)
    d2 = sq[:, None] + sq[None, :] - 2.0 * (x @ x.T)
    mask = (d2 < cutoff * cutoff) & (~jnp.eye(n, dtype=bool))
    src, dst = jnp.nonzero(mask, size=max_edges, fill_value=0)
    valid_r = jnp.arange(max_edges) < jnp.sum(mask)
    seq_src = jnp.concatenate([jnp.arange(n - 1), jnp.arange(1, n)])
    seq_dst = jnp.concatenate([jnp.arange(1, n), jnp.arange(n - 1)])
    src = jnp.concatenate([src, seq_src])
    dst = jnp.concatenate([dst, seq_dst])
    valid = jnp.concatenate([valid_r, jnp.ones(2 * (n - 1), dtype=bool)])
    return src, dst, valid


def _egnn_ref(h, x, src, dst, valid, edge_attr, p):
    vm = valid.astype(h.dtype)[:, None]
    e_in = jnp.concatenate([h[src], h[dst], edge_attr], axis=-1)
    m = jax.nn.silu(e_in @ p['eW1'] + p['eb1'])
    m = jax.nn.silu(m @ p['eW2'] + p['eb2'])
    m = m * vm
    c = jax.nn.silu(m @ p['cW1'] + p['cb1']) @ p['cW2'] + p['cb2']
    trans = (x[dst] - x[src]) * c * vm
    agg_t = jax.ops.segment_sum(trans, dst, num_segments=x.shape[0])
    cnt = jax.ops.segment_sum(vm, dst, num_segments=x.shape[0])
    x_new = x + agg_t / (cnt + 1.0)
    agg_m = jax.ops.segment_sum(m, dst, num_segments=h.shape[0])
    h_new = h + jax.nn.silu(
        jnp.concatenate([h, agg_m], axis=-1) @ p['nW1'] + p['nb1']
    ) @ p['nW2'] + p['nb2']
    return h_new, x_new


def _safe_norm_ref(diff):
    return jnp.sqrt(jnp.sum(diff * diff, axis=-1) + 1e-12)


def kernel(z, residue_types, atom_types, residue_indices, atoms_per_residue,
           params):
    del atoms_per_residue
    n = z.shape[0]
    h = z @ params['latW'] + params['latb'] \
        + params['res_embed'][residue_types]
    x = jnp.zeros((n, 3), dtype=z.dtype).at[:, 0].set(
        jnp.arange(n, dtype=z.dtype) * 5.9)
    x = x - x.mean(axis=0, keepdims=True)

    for p in params['res_layers']:
        src, dst, valid = _residue_graph_ref(x, CUTOFF, EMAX_RES)
        edge_attr = _rbf_ref(_safe_norm_ref(x[src] - x[dst]), NUM_RBF, CUTOFF)
        h, x = _egnn_ref(h, x, src, dst, valid, edge_attr, p)

    atom_emb = params['atom_embed'][atom_types]
    res_f = h[residue_indices]
    comb = jnp.concatenate([res_f, atom_emb], axis=-1)
    r = jax.nn.silu(comb @ params['relW1'] + params['relb1'])
    r = jax.nn.silu(r @ params['relW2'] + params['relb2'])
    rel = r @ params['relW3'] + params['relb3']
    atom_coords = x[residue_indices] + rel
    atom_features = res_f + atom_emb

    for p in params['atom_layers']:
        src, dst, valid = _atom_edges_sorted(atom_coords, ATOM_CUTOFF,
                                             EMAX_ATOM)
        atom_features, atom_coords = _egnn_layer_pallas(
            atom_features, atom_coords, src, dst, valid, p, ATOM_CUTOFF,
            N_ATOM)
    return atom_coords


# speedup vs baseline: 3.3266x; 3.3266x over previous
import functools

import jax
import jax.numpy as jnp
import numpy as np
from jax.experimental import pallas as pl
from jax.experimental.pallas import tpu as pltpu
from jax.experimental.pallas import tpu_sc as plsc

N_RES = 512
N_ATOM = 8192
HIDDEN = 128
NUM_RBF = 16
CUTOFF = 12.0
ATOM_CUTOFF = 5.0
EMAX_RES = 8192
EMAX_ATOM = 262144

EB = 256
WOUT = EB + 8
PCOLS = HIDDEN + 4


def _edge_block_kernel(rb8_ref, nvb_ref, ga_ref, gb_ref, xm_ref, w_ref,
                       out_ref, *, cutoff):
    b = pl.program_id(0)

    @pl.when(b == 0)
    def _init():
        out_ref[...] = jnp.zeros_like(out_ref)

    @pl.when(b < nvb_ref[0])
    def _compute():
        a = ga_ref[...]
        bv = gb_ref[...]
        xm = xm_ref[...]
        xs = xm[:, 0:3]
        xd = xm[:, 3:6]
        vm = xm[:, 6:7]
        lid = xm[:, 7:8]

        diff = xd - xs
        d = jnp.sqrt(jnp.sum(diff * diff, axis=1, keepdims=True) + 1e-12)
        centers = jax.lax.broadcasted_iota(
            jnp.int32, (1, NUM_RBF), 1).astype(jnp.float32) \
            * np.float32(cutoff / (NUM_RBF - 1))
        width = cutoff / NUM_RBF
        rbf = jnp.exp(-((d - centers) ** 2) / (2.0 * width * width))

        pre = a + bv + jnp.dot(rbf, w_ref[0:NUM_RBF, :],
                               preferred_element_type=jnp.float32)
        m1 = jax.nn.silu(pre)
        m = jax.nn.silu(
            jnp.dot(m1, w_ref[16:144, :], preferred_element_type=jnp.float32)
            + w_ref[272:273, :]) * vm
        s = jax.nn.silu(
            jnp.dot(m, w_ref[144:272, :], preferred_element_type=jnp.float32)
            + w_ref[273:274, :])
        c = jnp.sum(s * w_ref[274:275, :], axis=1, keepdims=True) \
            + w_ref[275:276, 0:1]
        trans = diff * (c * vm)

        p = jnp.concatenate([m, trans, vm], axis=1)
        sel = (lid == jax.lax.broadcasted_iota(jnp.int32, (EB, WOUT), 1)
               .astype(jnp.float32)).astype(jnp.float32)
        l = jax.lax.dot_general(sel, p, (((0,), (0,)), ((), ())),
                                preferred_element_type=jnp.float32)
        rb = pl.multiple_of(rb8_ref[b], 8)
        out_ref[pl.ds(rb, WOUT), :] += l


def _run_edge_kernel(gab, xm, wpack, rb8, nvb, nseg_pad, cutoff):
    epad = xm.shape[0]
    nb = epad // EB
    grid_spec = pltpu.PrefetchScalarGridSpec(
        num_scalar_prefetch=2,
        grid=(nb,),
        in_specs=[
            pl.BlockSpec((EB, HIDDEN), lambda b, rb8, nvb: (b, 0)),
            pl.BlockSpec((EB, HIDDEN), lambda b, rb8, nvb, _nb=nb: (_nb + b, 0)),
            pl.BlockSpec((EB, 8), lambda b, rb8, nvb: (b, 0)),
            pl.BlockSpec(wpack.shape, lambda b, rb8, nvb: (0, 0)),
        ],
        out_specs=pl.BlockSpec((nseg_pad, PCOLS), lambda b, rb8, nvb: (0, 0)),
    )
    return pl.pallas_call(
        functools.partial(_edge_block_kernel, cutoff=cutoff),
        grid_spec=grid_spec,
        out_shape=jax.ShapeDtypeStruct((nseg_pad, PCOLS), jnp.float32),
    )(rb8, nvb, gab, gab, xm, wpack)


_GW = 128


def _sc_gather_rows(table, indices):
    num_idx = indices.shape[0]
    cols = table.shape[1]
    idx2 = indices.reshape(1, num_idx)
    mesh = plsc.VectorSubcoreMesh(core_axis_name="core",
                                  subcore_axis_name="subcore")

    @functools.partial(pl.kernel,
                       out_type=jax.ShapeDtypeStruct((num_idx, cols),
                                                     table.dtype),
                       mesh=mesh)
    def kern(x_hbm, i_hbm, o_hbm):
        def body(i_vmem, o_vmem):
            pltpu.sync_copy(x_hbm.at[i_vmem.at[0]], o_vmem)

        pltpu.emit_pipeline(
            body,
            grid=(num_idx // _GW,),
            in_specs=[pl.BlockSpec((1, _GW), index_map=lambda i: (0, i))],
            out_specs=[pl.BlockSpec((_GW, cols), index_map=lambda i: (i, 0))],
            core_axis_name="subcore",
            dimension_semantics=(pltpu.PARALLEL,),
        )(i_hbm, o_hbm)

    return kern(table, idx2)


def _pack_weights(p):
    cb2 = jnp.broadcast_to(p['cb2'].reshape(1, 1), (1, HIDDEN))
    return jnp.concatenate([
        p['eW1'][2 * HIDDEN:],
        p['eW2'],
        p['cW1'],
        p['eb2'].reshape(1, HIDDEN),
        p['cb1'].reshape(1, HIDDEN),
        p['cW2'].reshape(1, HIDDEN),
        cb2,
        jnp.zeros((4, HIDDEN), jnp.float32),
    ], axis=0)


def _egnn_layer_pallas(h, x, src, dst, valid, p, cutoff, n):
    epad = src.shape[0]
    nb = epad // EB
    nseg_pad = n + WOUT + 8

    a = h @ p['eW1'][:HIDDEN]
    bn = h @ p['eW1'][HIDDEN:2 * HIDDEN] + p['eb1']
    vm = valid.astype(jnp.float32)

    dst32 = dst.astype(jnp.int32)
    is_start = jnp.concatenate([
        jnp.ones((1,), jnp.int32),
        (dst32[1:] != dst32[:-1]).astype(jnp.int32)])
    gseg = jnp.cumsum(is_start) - 1
    row_base = gseg.reshape(nb, EB)[:, 0]
    rb8 = ((row_base // 8) * 8).astype(jnp.int32)
    lid = (gseg.reshape(nb, EB) - rb8[:, None]).reshape(epad)
    seg_rows = jnp.zeros((n,), jnp.int32).at[gseg].set(dst32, mode='drop')
    nvalid = jnp.sum(vm).astype(jnp.int32)
    nvb = ((nvalid + EB - 1) // EB).reshape(1)

    table = jnp.concatenate([a, bn], axis=0)
    idx_all = jnp.concatenate([src, dst + n]).astype(jnp.int32)
    gab = _sc_gather_rows(table, idx_all)
    xm = jnp.concatenate([
        x[src], x[dst], vm[:, None], lid.astype(jnp.float32)[:, None]],
        axis=1)

    out_seg = _run_edge_kernel(gab, xm, _pack_weights(p), rb8, nvb, nseg_pad,
                               cutoff)
    agg = jnp.zeros((n, PCOLS), jnp.float32).at[seg_rows].add(out_seg[:n])

    agg_m = agg[:, :HIDDEN]
    agg_t = agg[:, HIDDEN:HIDDEN + 3]
    cnt = agg[:, HIDDEN + 3:HIDDEN + 4]
    x_new = x + agg_t / (cnt + 1.0)
    h_new = h + jax.nn.silu(
        jnp.concatenate([h, agg_m], axis=1) @ p['nW1'] + p['nb1']
    ) @ p['nW2'] + p['nb2']
    return h_new, x_new


_SCH = 2048
_DCH = 128


def _rank_block_kernel(mT_ref, offs_ref, idx_ref, carry_ref, *, emax, n):
    i = pl.program_id(1)

    @pl.when(i == 0)
    def _init():
        carry_ref[...] = jnp.zeros_like(carry_ref)

    m = mT_ref[...].astype(jnp.int32)
    inc = m
    k = 1
    while k < _SCH:
        inc = inc + jnp.concatenate(
            [jnp.zeros((k, _DCH), jnp.int32), inc[:_SCH - k]], axis=0)
        k *= 2
    carry = carry_ref[0:1, :]
    offs = offs_ref[0, 0:1, :]
    idx_ref[...] = offs + carry + inc
    carry_ref[0:1, :] = carry + inc[_SCH - 1:_SCH, :]


def _rank_compact(mT8, offs, emax):
    n = mT8.shape[0]
    offs3 = offs.reshape(n // _DCH, 1, _DCH)
    return pl.pallas_call(
        functools.partial(_rank_block_kernel, emax=emax, n=n),
        grid=(n // _DCH, n // _SCH),
        in_specs=[
            pl.BlockSpec((_SCH, _DCH), lambda j, i: (i, j)),
            pl.BlockSpec((1, 1, _DCH), lambda j, i: (j, 0, 0)),
        ],
        out_specs=pl.BlockSpec((_SCH, _DCH), lambda j, i: (i, j)),
        out_shape=jax.ShapeDtypeStruct((n, n), jnp.int32),
        scratch_shapes=[pltpu.VMEM((8, _DCH), jnp.int32)],
    )(mT8, offs3)


def _atom_edges_sorted(x, cutoff, emax):
    n = x.shape[0]
    sq = jnp.sum(x * x, axis=1)
    d2 = sq[:, None] + sq[None, :] - 2.0 * (x @ x.T)
    mask = (d2 < cutoff * cutoff) & (~jnp.eye(n, dtype=bool))
    cnt = jnp.sum(mask, axis=1, dtype=jnp.int32)
    offs = (jnp.cumsum(cnt) - cnt).astype(jnp.int32)
    c = _rank_compact(mask.T.astype(jnp.int8), offs, emax)
    flat_q = jnp.cumsum(jnp.bincount(c.reshape(-1), length=emax))
    total = jnp.sum(mask)
    valid = jnp.arange(emax) < total
    dstv = jnp.where(valid, flat_q // n, n - 1).astype(jnp.int32)
    srcv = jnp.where(valid, flat_q % n, n - 1).astype(jnp.int32)
    return srcv, dstv, valid


def _rbf_ref(d, num_rbf, cutoff):
    centers = jnp.linspace(0.0, cutoff, num_rbf)
    width = cutoff / num_rbf
    return jnp.exp(-((d[:, None] - centers[None, :]) ** 2)
                   / (2.0 * width * width))


def _residue_graph_ref(x, cutoff, max_edges):
    n = x.shape[0]
    sq = jnp.sum(x * x, axis=1)
    d2 = sq[:, None] + sq[None, :] - 2.0 * (x @ x.T)
    mask = (d2 < cutoff * cutoff) & (~jnp.eye(n, dtype=bool))
    src, dst = jnp.nonzero(mask, size=max_edges, fill_value=0)
    valid_r = jnp.arange(max_edges) < jnp.sum(mask)
    seq_src = jnp.concatenate([jnp.arange(n - 1), jnp.arange(1, n)])
    seq_dst = jnp.concatenate([jnp.arange(1, n), jnp.arange(n - 1)])
    src = jnp.concatenate([src, seq_src])
    dst = jnp.concatenate([dst, seq_dst])
    valid = jnp.concatenate([valid_r, jnp.ones(2 * (n - 1), dtype=bool)])
    return src, dst, valid


def _egnn_ref(h, x, src, dst, valid, edge_attr, p):
    vm = valid.astype(h.dtype)[:, None]
    e_in = jnp.concatenate([h[src], h[dst], edge_attr], axis=-1)
    m = jax.nn.silu(e_in @ p['eW1'] + p['eb1'])
    m = jax.nn.silu(m @ p['eW2'] + p['eb2'])
    m = m * vm
    c = jax.nn.silu(m @ p['cW1'] + p['cb1']) @ p['cW2'] + p['cb2']
    trans = (x[dst] - x[src]) * c * vm
    agg_t = jax.ops.segment_sum(trans, dst, num_segments=x.shape[0])
    cnt = jax.ops.segment_sum(vm, dst, num_segments=x.shape[0])
    x_new = x + agg_t / (cnt + 1.0)
    agg_m = jax.ops.segment_sum(m, dst, num_segments=h.shape[0])
    h_new = h + jax.nn.silu(
        jnp.concatenate([h, agg_m], axis=-1) @ p['nW1'] + p['nb1']
    ) @ p['nW2'] + p['nb2']
    return h_new, x_new


def _safe_norm_ref(diff):
    return jnp.sqrt(jnp.sum(diff * diff, axis=-1) + 1e-12)


def kernel(z, residue_types, atom_types, residue_indices, atoms_per_residue,
           params):
    del atoms_per_residue
    n = z.shape[0]
    h = z @ params['latW'] + params['latb'] \
        + params['res_embed'][residue_types]
    x = jnp.zeros((n, 3), dtype=z.dtype).at[:, 0].set(
        jnp.arange(n, dtype=z.dtype) * 5.9)
    x = x - x.mean(axis=0, keepdims=True)

    for p in params['res_layers']:
        src, dst, valid = _residue_graph_ref(x, CUTOFF, EMAX_RES)
        edge_attr = _rbf_ref(_safe_norm_ref(x[src] - x[dst]), NUM_RBF, CUTOFF)
        h, x = _egnn_ref(h, x, src, dst, valid, edge_attr, p)

    atom_emb = params['atom_embed'][atom_types]
    res_f = h[residue_indices]
    comb = jnp.concatenate([res_f, atom_emb], axis=-1)
    r = jax.nn.silu(comb @ params['relW1'] + params['relb1'])
    r = jax.nn.silu(r @ params['relW2'] + params['relb2'])
    rel = r @ params['relW3'] + params['relb3']
    atom_coords = x[residue_indices] + rel
    atom_features = res_f + atom_emb

    for p in params['atom_layers']:
        src, dst, valid = _atom_edges_sorted(atom_coords, ATOM_CUTOFF,
                                             EMAX_ATOM)
        atom_features, atom_coords = _egnn_layer_pallas(
            atom_features, atom_coords, src, dst, valid, p, ATOM_CUTOFF,
            N_ATOM)
    return atom_coords


# speedup vs baseline: 3.8655x; 1.1620x over previous
import functools

import jax
import jax.numpy as jnp
import numpy as np
from jax.experimental import pallas as pl
from jax.experimental.pallas import tpu as pltpu
from jax.experimental.pallas import tpu_sc as plsc

N_RES = 512
N_ATOM = 8192
HIDDEN = 128
NUM_RBF = 16
CUTOFF = 12.0
ATOM_CUTOFF = 5.0
EMAX_RES = 8192
EMAX_ATOM = 262144

EB = 256
WOUT = EB + 8
PCOLS = HIDDEN + 4


def _edge_block_kernel(rb8_ref, nvb_ref, ga_ref, gb_ref, xm_ref, w_ref,
                       out_ref, *, cutoff):
    b = pl.program_id(0)

    @pl.when(b == 0)
    def _init():
        out_ref[...] = jnp.zeros_like(out_ref)

    @pl.when(b < nvb_ref[0])
    def _compute():
        a = ga_ref[...]
        bv = gb_ref[...]
        xm = xm_ref[...]
        xs = xm[:, 0:3]
        xd = xm[:, 3:6]
        vm = xm[:, 6:7]
        lid = xm[:, 7:8]

        diff = xd - xs
        d = jnp.sqrt(jnp.sum(diff * diff, axis=1, keepdims=True) + 1e-12)
        centers = jax.lax.broadcasted_iota(
            jnp.int32, (1, NUM_RBF), 1).astype(jnp.float32) \
            * np.float32(cutoff / (NUM_RBF - 1))
        width = cutoff / NUM_RBF
        rbf = jnp.exp(-((d - centers) ** 2) / (2.0 * width * width))

        pre = a + bv + jnp.dot(rbf, w_ref[0:NUM_RBF, :],
                               preferred_element_type=jnp.float32)
        m1 = jax.nn.silu(pre)
        m = jax.nn.silu(
            jnp.dot(m1, w_ref[16:144, :], preferred_element_type=jnp.float32)
            + w_ref[272:273, :]) * vm
        s = jax.nn.silu(
            jnp.dot(m, w_ref[144:272, :], preferred_element_type=jnp.float32)
            + w_ref[273:274, :])
        c = jnp.sum(s * w_ref[274:275, :], axis=1, keepdims=True) \
            + w_ref[275:276, 0:1]
        trans = diff * (c * vm)

        p = jnp.concatenate([m, trans, vm], axis=1)
        sel = (lid == jax.lax.broadcasted_iota(jnp.int32, (EB, WOUT), 1)
               .astype(jnp.float32)).astype(jnp.float32)
        l = jax.lax.dot_general(sel, p, (((0,), (0,)), ((), ())),
                                preferred_element_type=jnp.float32)
        rb = pl.multiple_of(rb8_ref[b], 8)
        out_ref[pl.ds(rb, WOUT), :] += l


def _run_edge_kernel(gab, xm, wpack, rb8, nvb, nseg_pad, cutoff):
    epad = xm.shape[0]
    nb = epad // EB
    grid_spec = pltpu.PrefetchScalarGridSpec(
        num_scalar_prefetch=2,
        grid=(nb,),
        in_specs=[
            pl.BlockSpec((EB, HIDDEN), lambda b, rb8, nvb: (b, 0)),
            pl.BlockSpec((EB, HIDDEN), lambda b, rb8, nvb, _nb=nb: (_nb + b, 0)),
            pl.BlockSpec((EB, 8), lambda b, rb8, nvb: (b, 0)),
            pl.BlockSpec(wpack.shape, lambda b, rb8, nvb: (0, 0)),
        ],
        out_specs=pl.BlockSpec((nseg_pad, PCOLS), lambda b, rb8, nvb: (0, 0)),
    )
    return pl.pallas_call(
        functools.partial(_edge_block_kernel, cutoff=cutoff),
        grid_spec=grid_spec,
        out_shape=jax.ShapeDtypeStruct((nseg_pad, PCOLS), jnp.float32),
    )(rb8, nvb, gab, gab, xm, wpack)


_GW = 128


def _sc_gather_rows(table, indices):
    num_idx = indices.shape[0]
    cols = table.shape[1]
    idx2 = indices.reshape(1, num_idx)
    mesh = plsc.VectorSubcoreMesh(core_axis_name="core",
                                  subcore_axis_name="subcore")

    @functools.partial(pl.kernel,
                       out_type=jax.ShapeDtypeStruct((num_idx, cols),
                                                     table.dtype),
                       mesh=mesh)
    def kern(x_hbm, i_hbm, o_hbm):
        def body(i_vmem, o_vmem):
            pltpu.sync_copy(x_hbm.at[i_vmem.at[0]], o_vmem)

        pltpu.emit_pipeline(
            body,
            grid=(num_idx // _GW,),
            in_specs=[pl.BlockSpec((1, _GW), index_map=lambda i: (0, i))],
            out_specs=[pl.BlockSpec((_GW, cols), index_map=lambda i: (i, 0))],
            core_axis_name="subcore",
            dimension_semantics=(pltpu.PARALLEL,),
        )(i_hbm, o_hbm)

    return kern(table, idx2)


def _pack_weights(p):
    cb2 = jnp.broadcast_to(p['cb2'].reshape(1, 1), (1, HIDDEN))
    return jnp.concatenate([
        p['eW1'][2 * HIDDEN:],
        p['eW2'],
        p['cW1'],
        p['eb2'].reshape(1, HIDDEN),
        p['cb1'].reshape(1, HIDDEN),
        p['cW2'].reshape(1, HIDDEN),
        cb2,
        jnp.zeros((4, HIDDEN), jnp.float32),
    ], axis=0)


def _egnn_layer_pallas(h, x, src, dst, valid, cnt_node, p, cutoff, n):
    epad = src.shape[0]
    nb = epad // EB
    nseg_pad = n + WOUT + 8

    a = h @ p['eW1'][:HIDDEN]
    bn = h @ p['eW1'][HIDDEN:2 * HIDDEN] + p['eb1']
    vm = valid.astype(jnp.float32)

    dst32 = dst.astype(jnp.int32)
    is_start = jnp.concatenate([
        jnp.ones((1,), jnp.int32),
        (dst32[1:] != dst32[:-1]).astype(jnp.int32)])
    gseg = jnp.cumsum(is_start) - 1
    row_base = gseg.reshape(nb, EB)[:, 0]
    rb8 = ((row_base // 8) * 8).astype(jnp.int32)
    lid = (gseg.reshape(nb, EB) - rb8[:, None]).reshape(epad)
    nvalid = jnp.sum(vm).astype(jnp.int32)
    nvb = ((nvalid + EB - 1) // EB).reshape(1)

    table = jnp.concatenate([a, bn], axis=0)
    idx_all = jnp.concatenate([src, dst + n]).astype(jnp.int32)
    gab = _sc_gather_rows(table, idx_all)
    xm = jnp.concatenate([
        x[src], x[dst], vm[:, None], lid.astype(jnp.float32)[:, None]],
        axis=1)

    out_seg = _run_edge_kernel(gab, xm, _pack_weights(p), rb8, nvb, nseg_pad,
                               cutoff)
    node_has = cnt_node > 0
    node_gid = jnp.cumsum(node_has.astype(jnp.int32)) - 1
    agg = jnp.where(node_has[:, None], out_seg[node_gid], 0.0)

    agg_m = agg[:, :HIDDEN]
    agg_t = agg[:, HIDDEN:HIDDEN + 3]
    cnt = agg[:, HIDDEN + 3:HIDDEN + 4]
    x_new = x + agg_t / (cnt + 1.0)
    h_new = h + jax.nn.silu(
        jnp.concatenate([h, agg_m], axis=1) @ p['nW1'] + p['nb1']
    ) @ p['nW2'] + p['nb2']
    return h_new, x_new


_SCH = 2048
_DCH = 128


def _rank_block_kernel(mT_ref, offs_ref, idx_ref, carry_ref, *, emax, n):
    i = pl.program_id(1)

    @pl.when(i == 0)
    def _init():
        carry_ref[...] = jnp.zeros_like(carry_ref)

    m = jnp.transpose(mT_ref[...].astype(jnp.int32))
    inc = m
    k = 1
    while k < _SCH:
        inc = inc + jnp.concatenate(
            [jnp.zeros((k, _DCH), jnp.int32), inc[:_SCH - k]], axis=0)
        k *= 2
    carry = carry_ref[0:1, :]
    offs = offs_ref[0, 0:1, :]
    idx_ref[...] = offs + carry + inc
    carry_ref[0:1, :] = carry + inc[_SCH - 1:_SCH, :]


def _rank_compact(m8, offs, emax):
    n = m8.shape[0]
    offs3 = offs.reshape(n // _DCH, 1, _DCH)
    return pl.pallas_call(
        functools.partial(_rank_block_kernel, emax=emax, n=n),
        grid=(n // _DCH, n // _SCH),
        in_specs=[
            pl.BlockSpec((_DCH, _SCH), lambda j, i: (j, i)),
            pl.BlockSpec((1, 1, _DCH), lambda j, i: (j, 0, 0)),
        ],
        out_specs=pl.BlockSpec((_SCH, _DCH), lambda j, i: (i, j)),
        out_shape=jax.ShapeDtypeStruct((n, n), jnp.int32),
        scratch_shapes=[pltpu.VMEM((8, _DCH), jnp.int32)],
    )(m8, offs3)


def _atom_edges_sorted(x, cutoff, emax):
    n = x.shape[0]
    sq = jnp.sum(x * x, axis=1)
    d2 = sq[:, None] + sq[None, :] - 2.0 * (x @ x.T)
    mask = (d2 < cutoff * cutoff) & (~jnp.eye(n, dtype=bool))
    cnt = jnp.sum(mask, axis=1, dtype=jnp.int32)
    offs = (jnp.cumsum(cnt) - cnt).astype(jnp.int32)
    c = _rank_compact(mask.astype(jnp.int8), offs, emax)
    flat_q = jnp.cumsum(jnp.bincount(c.reshape(-1), length=emax))
    total = jnp.sum(mask)
    valid = jnp.arange(emax) < total
    key = jnp.where(valid, (flat_q % n).astype(jnp.int32),
                    jnp.int32(n)).astype(jnp.int32)
    key_s, q_s = jax.lax.sort((key, flat_q.astype(jnp.int32)), num_keys=1)
    valid = key_s < n
    srcv = jnp.where(valid, q_s // n, n - 1).astype(jnp.int32)
    dstv = jnp.where(valid, q_s % n, n - 1).astype(jnp.int32)
    cnt_trunc = jnp.bincount(jnp.where(valid, dstv, n), length=n)
    return srcv, dstv, valid, cnt_trunc.astype(jnp.int32)


def _rbf_ref(d, num_rbf, cutoff):
    centers = jnp.linspace(0.0, cutoff, num_rbf)
    width = cutoff / num_rbf
    return jnp.exp(-((d[:, None] - centers[None, :]) ** 2)
                   / (2.0 * width * width))


def _residue_graph_ref(x, cutoff, max_edges):
    n = x.shape[0]
    sq = jnp.sum(x * x, axis=1)
    d2 = sq[:, None] + sq[None, :] - 2.0 * (x @ x.T)
    mask = (d2 < cutoff * cutoff) & (~jnp.eye(n, dtype=bool))
    src, dst = jnp.nonzero(mask, size=max_edges, fill_value=0)
    valid_r = jnp.arange(max_edges) < jnp.sum(mask)
    seq_src = jnp.concatenate([jnp.arange(n - 1), jnp.arange(1, n)])
    seq_dst = jnp.concatenate([jnp.arange(1, n), jnp.arange(n - 1)])
    src = jnp.concatenate([src, seq_src])
    dst = jnp.concatenate([dst, seq_dst])
    valid = jnp.concatenate([valid_r, jnp.ones(2 * (n - 1), dtype=bool)])
    return src, dst, valid


def _egnn_ref(h, x, src, dst, valid, edge_attr, p):
    vm = valid.astype(h.dtype)[:, None]
    e_in = jnp.concatenate([h[src], h[dst], edge_attr], axis=-1)
    m = jax.nn.silu(e_in @ p['eW1'] + p['eb1'])
    m = jax.nn.silu(m @ p['eW2'] + p['eb2'])
    m = m * vm
    c = jax.nn.silu(m @ p['cW1'] + p['cb1']) @ p['cW2'] + p['cb2']
    trans = (x[dst] - x[src]) * c * vm
    agg_t = jax.ops.segment_sum(trans, dst, num_segments=x.shape[0])
    cnt = jax.ops.segment_sum(vm, dst, num_segments=x.shape[0])
    x_new = x + agg_t / (cnt + 1.0)
    agg_m = jax.ops.segment_sum(m, dst, num_segments=h.shape[0])
    h_new = h + jax.nn.silu(
        jnp.concatenate([h, agg_m], axis=-1) @ p['nW1'] + p['nb1']
    ) @ p['nW2'] + p['nb2']
    return h_new, x_new


def _safe_norm_ref(diff):
    return jnp.sqrt(jnp.sum(diff * diff, axis=-1) + 1e-12)


def kernel(z, residue_types, atom_types, residue_indices, atoms_per_residue,
           params):
    del atoms_per_residue
    n = z.shape[0]
    h = z @ params['latW'] + params['latb'] \
        + params['res_embed'][residue_types]
    x = jnp.zeros((n, 3), dtype=z.dtype).at[:, 0].set(
        jnp.arange(n, dtype=z.dtype) * 5.9)
    x = x - x.mean(axis=0, keepdims=True)

    for p in params['res_layers']:
        src, dst, valid = _residue_graph_ref(x, CUTOFF, EMAX_RES)
        edge_attr = _rbf_ref(_safe_norm_ref(x[src] - x[dst]), NUM_RBF, CUTOFF)
        h, x = _egnn_ref(h, x, src, dst, valid, edge_attr, p)

    atom_emb = params['atom_embed'][atom_types]
    res_f = h[residue_indices]
    comb = jnp.concatenate([res_f, atom_emb], axis=-1)
    r = jax.nn.silu(comb @ params['relW1'] + params['relb1'])
    r = jax.nn.silu(r @ params['relW2'] + params['relb2'])
    rel = r @ params['relW3'] + params['relb3']
    atom_coords = x[residue_indices] + rel
    atom_features = res_f + atom_emb

    for p in params['atom_layers']:
        src, dst, valid, cnt_node = _atom_edges_sorted(
            atom_coords, ATOM_CUTOFF, EMAX_ATOM)
        atom_features, atom_coords = _egnn_layer_pallas(
            atom_features, atom_coords, src, dst, valid, cnt_node, p,
            ATOM_CUTOFF, N_ATOM)
    return atom_coords
